# baseline (speedup 1.0000x reference)
import functools

import jax
import jax.numpy as jnp
from jax import lax
from jax.experimental import pallas as pl
from jax.experimental.pallas import tpu as pltpu
from jax.experimental.pallas import tpu_sc as plsc

N = 10000
E = 320000
D = 64
NW = 32
CHUNK = 128
EPW = 10240
NCHUNK = EPW // CHUNK
NPAD = 10240
RPT = NPAD // 16
DEGW = 16

M0 = 38
M1 = 122
MMAX = max(M0, M1)
NTOT = 16 * (M0 + M1)
NFLAT = NTOT + MMAX

_SC_CACHE = {}


def _build_deg_kernel():
    if "deg" in _SC_CACHE:
        return _SC_CACHE["deg"]
    kern = functools.partial(
        pl.kernel,
        out_type=jax.ShapeDtypeStruct((2, NPAD, DEGW), jnp.float32),
        mesh=plsc.VectorSubcoreMesh(core_axis_name="c", subcore_axis_name="s"),
        scratch_types=[
            pltpu.VMEM((NCHUNK, CHUNK), jnp.int32),
            pltpu.VMEM((CHUNK, DEGW), jnp.float32),
            pltpu.VMEM_SHARED((NPAD, DEGW), jnp.float32),
            pltpu.SemaphoreType.DMA,
        ],
    )(_deg_body)
    _SC_CACHE["deg"] = kern
    return kern


def _deg_body(dst_hbm, out_hbm, dst_v, ones_v, acc_sh, sem):
    cid = lax.axis_index("c")
    sid = lax.axis_index("s")
    wid = cid * 16 + sid
    cp = pltpu.async_copy(dst_hbm.at[wid], dst_v, sem)

    def fill(val):
        def body(i, carry):
            ones_v[i, pl.ds(0, 16)] = jnp.full((16,), val, jnp.float32)
            return carry
        lax.fori_loop(0, CHUNK, body, 0)

    fill(0.0)
    base = sid * RPT
    for r in range(RPT // CHUNK):
        pltpu.sync_copy(ones_v, acc_sh.at[pl.ds(base + r * CHUNK, CHUNK)])
    fill(1.0)
    cp.wait()
    plsc.subcore_barrier()

    def body(j, carry):
        pltpu.sync_copy(ones_v, acc_sh.at[dst_v.at[j]], add=True)
        return carry
    lax.fori_loop(0, NCHUNK, body, 0)

    plsc.subcore_barrier()
    pltpu.sync_copy(acc_sh.at[pl.ds(base, RPT)], out_hbm.at[cid, pl.ds(base, RPT)])


def _build_agg_kernel():
    if "agg" in _SC_CACHE:
        return _SC_CACHE["agg"]
    kern = functools.partial(
        pl.kernel,
        out_type=jax.ShapeDtypeStruct((2, NPAD, D), jnp.float32),
        mesh=plsc.VectorSubcoreMesh(core_axis_name="c", subcore_axis_name="s"),
        scratch_types=[
            pltpu.VMEM((MMAX, CHUNK), jnp.int32),
            pltpu.VMEM((MMAX, CHUNK), jnp.int32),
            pltpu.VMEM((2 * CHUNK, D), jnp.float32),
            pltpu.VMEM_SHARED((NPAD, D), jnp.float32),
            pltpu.SemaphoreType.DMA,
            pltpu.SemaphoreType.DMA,
            pltpu.SemaphoreType.DMA,
        ],
        compiler_params=pltpu.CompilerParams(use_tc_tiling_on_sc=False),
    )(_agg_body)
    _SC_CACHE["agg"] = kern
    return kern


def _agg_body(g_hbm, src_hbm, dst_hbm, out_hbm,
              src_v, dst_v, rows_v, acc_sh, g0, s0, s1):
    cid = lax.axis_index("c")
    sid = lax.axis_index("s")
    wid = cid * 16 + sid
    cnt = jnp.where(cid == 0, M0, M1)
    cp_s = pltpu.async_copy(src_hbm.at[wid], src_v, s0)
    cp_d = pltpu.async_copy(dst_hbm.at[wid], dst_v, s1)

    def zbody(i, carry):
        for c in range(D // 16):
            rows_v[i, pl.ds(c * 16, 16)] = jnp.zeros((16,), jnp.float32)
        return carry
    lax.fori_loop(0, CHUNK, zbody, 0)

    base = sid * RPT
    for r in range(RPT // CHUNK):
        pltpu.sync_copy(rows_v.at[pl.ds(0, CHUNK)],
                        acc_sh.at[pl.ds(base + r * CHUNK, CHUNK)])
    cp_s.wait()
    cp_d.wait()
    plsc.subcore_barrier()

    def body(j, carry):
        goff = lax.rem(j, 2) * CHUNK
        jp = jnp.maximum(j - 1, 0)
        poff = lax.rem(jp, 2) * CHUNK

        @pl.when(j < cnt)
        def _():
            pltpu.async_copy(
                g_hbm.at[src_v.at[j]], rows_v.at[pl.ds(goff, CHUNK)], g0)

        @pl.when(j > 0)
        def _():
            pltpu.make_async_copy(
                g_hbm.at[src_v.at[jp]],
                rows_v.at[pl.ds(poff, CHUNK)], g0).wait()
            pltpu.sync_copy(rows_v.at[pl.ds(poff, CHUNK)],
                            acc_sh.at[dst_v.at[jp]], add=True)
        return carry
    lax.fori_loop(0, cnt + 1, body, 0)

    plsc.subcore_barrier()
    pltpu.sync_copy(acc_sh.at[pl.ds(base, RPT)], out_hbm.at[cid, pl.ds(base, RPT)])


def _mm_body(x_ref, w_ref, o_ref):
    o_ref[...] = jnp.dot(x_ref[...], w_ref[...],
                         preferred_element_type=jnp.float32)


def _matmul(x, w, bm=1000):
    m, k = x.shape
    n = w.shape[1]
    return pl.pallas_call(
        _mm_body,
        grid=(m // bm,),
        in_specs=[pl.BlockSpec((bm, k), lambda i: (i, 0)),
                  pl.BlockSpec((k, n), lambda i: (0, 0))],
        out_specs=pl.BlockSpec((bm, n), lambda i: (i, 0)),
        out_shape=jax.ShapeDtypeStruct((m, n), jnp.float32),
    )(x, w)


def _scale0_body(h_ref, p0_ref, p1_ref, g_ref, dinv_ref):
    deg = p0_ref[...] + p1_ref[...] + 1.0
    dinv = lax.rsqrt(deg)
    dinv_ref[...] = dinv
    g_ref[...] = h_ref[...] * dinv


def _scale0(h, p0, p1, bm=1000):
    m = h.shape[0]
    return pl.pallas_call(
        _scale0_body,
        grid=(m // bm,),
        in_specs=[pl.BlockSpec((bm, D), lambda i: (i, 0)),
                  pl.BlockSpec((bm, 1), lambda i: (i, 0)),
                  pl.BlockSpec((bm, 1), lambda i: (i, 0))],
        out_specs=[pl.BlockSpec((bm, D), lambda i: (i, 0)),
                   pl.BlockSpec((bm, 1), lambda i: (i, 0))],
        out_shape=[jax.ShapeDtypeStruct((m, D), jnp.float32),
                   jax.ShapeDtypeStruct((m, 1), jnp.float32)],
    )(h, p0, p1)


def _mid_body(p0_ref, p1_ref, g_ref, dinv_ref, b_ref, w_ref, o_ref):
    dinv = dinv_ref[...]
    a = dinv * (p0_ref[...] + p1_ref[...] + g_ref[...]) + b_ref[...]
    a = jnp.maximum(a, 0.0)
    o_ref[...] = dinv * jnp.dot(a, w_ref[...],
                                preferred_element_type=jnp.float32)


def _mid(p0, p1, g, dinv, b, w, bm=1000):
    m = g.shape[0]
    n = w.shape[1]
    return pl.pallas_call(
        _mid_body,
        grid=(m // bm,),
        in_specs=[pl.BlockSpec((bm, D), lambda i: (i, 0)),
                  pl.BlockSpec((bm, D), lambda i: (i, 0)),
                  pl.BlockSpec((bm, D), lambda i: (i, 0)),
                  pl.BlockSpec((bm, 1), lambda i: (i, 0)),
                  pl.BlockSpec((1, D), lambda i: (0, 0)),
                  pl.BlockSpec((D, n), lambda i: (0, 0))],
        out_specs=pl.BlockSpec((bm, n), lambda i: (i, 0)),
        out_shape=jax.ShapeDtypeStruct((m, n), jnp.float32),
    )(p0, p1, g, dinv, b, w)


def _fin_body(p0_ref, p1_ref, g_ref, dinv_ref, b_ref, o_ref):
    o_ref[...] = (dinv_ref[...] * (p0_ref[...] + p1_ref[...] + g_ref[...])
                  + b_ref[...])


def _fin(p0, p1, g, dinv, b, bm=1000):
    m = g.shape[0]
    return pl.pallas_call(
        _fin_body,
        grid=(m // bm,),
        in_specs=[pl.BlockSpec((bm, D), lambda i: (i, 0)),
                  pl.BlockSpec((bm, D), lambda i: (i, 0)),
                  pl.BlockSpec((bm, D), lambda i: (i, 0)),
                  pl.BlockSpec((bm, 1), lambda i: (i, 0)),
                  pl.BlockSpec((1, D), lambda i: (0, 0))],
        out_specs=pl.BlockSpec((bm, D), lambda i: (i, 0)),
        out_shape=jax.ShapeDtypeStruct((m, D), jnp.float32),
    )(p0, p1, g, dinv, b)


def kernel(x, edge_index, W1, b1, W2, b2, W3, b3):
    ei = edge_index.astype(jnp.int32)
    src, dst = ei[0], ei[1]
    npad_e = NW * EPW - E
    dst3 = jnp.concatenate(
        [dst, jnp.full((npad_e,), N, jnp.int32)]).reshape(NW, NCHUNK, CHUNK)
    def arrange(vals, fill):
        slots = jnp.concatenate(
            [vals, jnp.full((NTOT * CHUNK - E,), fill, jnp.int32)]
        ).reshape(NTOT, CHUNK)
        a = slots[:16 * M0].reshape(16, M0, CHUNK)
        b = slots[16 * M0:].reshape(16, M1, CHUNK)
        a = jnp.concatenate(
            [a, jnp.full((16, MMAX - M0, CHUNK), fill, jnp.int32)], axis=1)
        b = jnp.concatenate(
            [b, jnp.full((16, MMAX - M1, CHUNK), fill, jnp.int32)], axis=1)
        return jnp.concatenate([a, b], axis=0)

    src_f = arrange(src, 0)
    dst_f = arrange(dst, N)

    pdeg = _build_deg_kernel()(dst3)
    p0 = pdeg[0, :N, 0:1]
    p1 = pdeg[1, :N, 0:1]

    h1 = _matmul(x, W1)
    g1, dinv = _scale0(h1, p0, p1)

    agg = _build_agg_kernel()
    acc1 = agg(g1, src_f, dst_f)
    g2 = _mid(acc1[0, :N], acc1[1, :N], g1, dinv, b1.reshape(1, D), W2)

    acc2 = agg(g2, src_f, dst_f)
    g3 = _mid(acc2[0, :N], acc2[1, :N], g2, dinv, b2.reshape(1, D), W3)

    acc3 = agg(g3, src_f, dst_f)
    return _fin(acc3[0, :N], acc3[1, :N], g3, dinv, b3.reshape(1, D))


# speedup vs baseline: 1.1463x; 1.1463x over previous
import functools

import jax
import jax.numpy as jnp
from jax import lax
from jax.experimental import pallas as pl
from jax.experimental.pallas import tpu as pltpu
from jax.experimental.pallas import tpu_sc as plsc

N = 10000
E = 320000
D = 64
NW = 32
CHUNK = 128
EPW = 10240
NCHUNK = EPW // CHUNK
NPAD = 10240
RPT = NPAD // 16
DEGW = 16

M0 = 122
M1 = 38
MMAX = max(M0, M1)
NTOT = 16 * (M0 + M1)
NFLAT = NTOT + MMAX

_SC_CACHE = {}


def _build_deg_kernel():
    if "deg" in _SC_CACHE:
        return _SC_CACHE["deg"]
    kern = functools.partial(
        pl.kernel,
        out_type=jax.ShapeDtypeStruct((2, NPAD, DEGW), jnp.float32),
        mesh=plsc.VectorSubcoreMesh(core_axis_name="c", subcore_axis_name="s"),
        scratch_types=[
            pltpu.VMEM((NCHUNK, CHUNK), jnp.int32),
            pltpu.VMEM((CHUNK, DEGW), jnp.float32),
            pltpu.VMEM_SHARED((NPAD, DEGW), jnp.float32),
            pltpu.SemaphoreType.DMA,
        ],
    )(_deg_body)
    _SC_CACHE["deg"] = kern
    return kern


def _deg_body(dst_hbm, out_hbm, dst_v, ones_v, acc_sh, sem):
    cid = lax.axis_index("c")
    sid = lax.axis_index("s")
    wid = cid * 16 + sid
    cp = pltpu.async_copy(dst_hbm.at[wid], dst_v, sem)

    def fill(val):
        def body(i, carry):
            ones_v[i, pl.ds(0, 16)] = jnp.full((16,), val, jnp.float32)
            return carry
        lax.fori_loop(0, CHUNK, body, 0)

    fill(0.0)
    base = sid * RPT
    for r in range(RPT // CHUNK):
        pltpu.sync_copy(ones_v, acc_sh.at[pl.ds(base + r * CHUNK, CHUNK)])
    fill(1.0)
    cp.wait()
    plsc.subcore_barrier()

    def body(j, carry):
        pltpu.sync_copy(ones_v, acc_sh.at[dst_v.at[j]], add=True)
        return carry
    lax.fori_loop(0, NCHUNK, body, 0)

    plsc.subcore_barrier()
    pltpu.sync_copy(acc_sh.at[pl.ds(base, RPT)], out_hbm.at[cid, pl.ds(base, RPT)])


def _build_agg_kernel():
    if "agg" in _SC_CACHE:
        return _SC_CACHE["agg"]
    kern = functools.partial(
        pl.kernel,
        out_type=jax.ShapeDtypeStruct((2, NPAD, D), jnp.float32),
        mesh=plsc.VectorSubcoreMesh(core_axis_name="c", subcore_axis_name="s"),
        scratch_types=[
            pltpu.VMEM((MMAX, CHUNK), jnp.int32),
            pltpu.VMEM((MMAX, CHUNK), jnp.int32),
            pltpu.VMEM((2 * CHUNK, D), jnp.float32),
            pltpu.VMEM_SHARED((NPAD, D), jnp.float32),
            pltpu.SemaphoreType.DMA,
            pltpu.SemaphoreType.DMA,
            pltpu.SemaphoreType.DMA,
        ],
        compiler_params=pltpu.CompilerParams(use_tc_tiling_on_sc=False),
    )(_agg_body)
    _SC_CACHE["agg"] = kern
    return kern


def _agg_body(g_hbm, src_hbm, dst_hbm, out_hbm,
              src_v, dst_v, rows_v, acc_sh, g0, s0, s1):
    cid = lax.axis_index("c")
    sid = lax.axis_index("s")
    wid = cid * 16 + sid
    cnt = jnp.where(cid == 0, M0, M1)
    cp_s = pltpu.async_copy(src_hbm.at[wid], src_v, s0)
    cp_d = pltpu.async_copy(dst_hbm.at[wid], dst_v, s1)

    def zbody(i, carry):
        for c in range(D // 16):
            rows_v[i, pl.ds(c * 16, 16)] = jnp.zeros((16,), jnp.float32)
        return carry
    lax.fori_loop(0, CHUNK, zbody, 0)

    base = sid * RPT
    for r in range(RPT // CHUNK):
        pltpu.sync_copy(rows_v.at[pl.ds(0, CHUNK)],
                        acc_sh.at[pl.ds(base + r * CHUNK, CHUNK)])
    cp_s.wait()
    cp_d.wait()
    plsc.subcore_barrier()

    def body(j, carry):
        goff = lax.rem(j, 2) * CHUNK
        jp = jnp.maximum(j - 1, 0)
        poff = lax.rem(jp, 2) * CHUNK

        @pl.when(j < cnt)
        def _():
            pltpu.async_copy(
                g_hbm.at[src_v.at[j]], rows_v.at[pl.ds(goff, CHUNK)], g0)

        @pl.when(j > 0)
        def _():
            pltpu.make_async_copy(
                g_hbm.at[src_v.at[jp]],
                rows_v.at[pl.ds(poff, CHUNK)], g0).wait()
            pltpu.sync_copy(rows_v.at[pl.ds(poff, CHUNK)],
                            acc_sh.at[dst_v.at[jp]], add=True)
        return carry
    lax.fori_loop(0, cnt + 1, body, 0)

    plsc.subcore_barrier()
    pltpu.sync_copy(acc_sh.at[pl.ds(base, RPT)], out_hbm.at[cid, pl.ds(base, RPT)])


def _mm_body(x_ref, w_ref, o_ref):
    o_ref[...] = jnp.dot(x_ref[...], w_ref[...],
                         preferred_element_type=jnp.float32)


def _matmul(x, w, bm=1000):
    m, k = x.shape
    n = w.shape[1]
    return pl.pallas_call(
        _mm_body,
        grid=(m // bm,),
        in_specs=[pl.BlockSpec((bm, k), lambda i: (i, 0)),
                  pl.BlockSpec((k, n), lambda i: (0, 0))],
        out_specs=pl.BlockSpec((bm, n), lambda i: (i, 0)),
        out_shape=jax.ShapeDtypeStruct((m, n), jnp.float32),
    )(x, w)


def _scale0_body(h_ref, p0_ref, p1_ref, g_ref, dinv_ref):
    deg = p0_ref[...] + p1_ref[...] + 1.0
    dinv = lax.rsqrt(deg)
    dinv_ref[...] = dinv
    g_ref[...] = h_ref[...] * dinv


def _scale0(h, p0, p1, bm=1000):
    m = h.shape[0]
    return pl.pallas_call(
        _scale0_body,
        grid=(m // bm,),
        in_specs=[pl.BlockSpec((bm, D), lambda i: (i, 0)),
                  pl.BlockSpec((bm, 1), lambda i: (i, 0)),
                  pl.BlockSpec((bm, 1), lambda i: (i, 0))],
        out_specs=[pl.BlockSpec((bm, D), lambda i: (i, 0)),
                   pl.BlockSpec((bm, 1), lambda i: (i, 0))],
        out_shape=[jax.ShapeDtypeStruct((m, D), jnp.float32),
                   jax.ShapeDtypeStruct((m, 1), jnp.float32)],
    )(h, p0, p1)


def _mid_body(p0_ref, p1_ref, g_ref, dinv_ref, b_ref, w_ref, o_ref):
    dinv = dinv_ref[...]
    a = dinv * (p0_ref[...] + p1_ref[...] + g_ref[...]) + b_ref[...]
    a = jnp.maximum(a, 0.0)
    o_ref[...] = dinv * jnp.dot(a, w_ref[...],
                                preferred_element_type=jnp.float32)


def _mid(p0, p1, g, dinv, b, w, bm=1000):
    m = g.shape[0]
    n = w.shape[1]
    return pl.pallas_call(
        _mid_body,
        grid=(m // bm,),
        in_specs=[pl.BlockSpec((bm, D), lambda i: (i, 0)),
                  pl.BlockSpec((bm, D), lambda i: (i, 0)),
                  pl.BlockSpec((bm, D), lambda i: (i, 0)),
                  pl.BlockSpec((bm, 1), lambda i: (i, 0)),
                  pl.BlockSpec((1, D), lambda i: (0, 0)),
                  pl.BlockSpec((D, n), lambda i: (0, 0))],
        out_specs=pl.BlockSpec((bm, n), lambda i: (i, 0)),
        out_shape=jax.ShapeDtypeStruct((m, n), jnp.float32),
    )(p0, p1, g, dinv, b, w)


def _fin_body(p0_ref, p1_ref, g_ref, dinv_ref, b_ref, o_ref):
    o_ref[...] = (dinv_ref[...] * (p0_ref[...] + p1_ref[...] + g_ref[...])
                  + b_ref[...])


def _fin(p0, p1, g, dinv, b, bm=1000):
    m = g.shape[0]
    return pl.pallas_call(
        _fin_body,
        grid=(m // bm,),
        in_specs=[pl.BlockSpec((bm, D), lambda i: (i, 0)),
                  pl.BlockSpec((bm, D), lambda i: (i, 0)),
                  pl.BlockSpec((bm, D), lambda i: (i, 0)),
                  pl.BlockSpec((bm, 1), lambda i: (i, 0)),
                  pl.BlockSpec((1, D), lambda i: (0, 0))],
        out_specs=pl.BlockSpec((bm, D), lambda i: (i, 0)),
        out_shape=jax.ShapeDtypeStruct((m, D), jnp.float32),
    )(p0, p1, g, dinv, b)


def kernel(x, edge_index, W1, b1, W2, b2, W3, b3):
    ei = edge_index.astype(jnp.int32)
    src, dst = ei[0], ei[1]
    npad_e = NW * EPW - E
    dst3 = jnp.concatenate(
        [dst, jnp.full((npad_e,), N, jnp.int32)]).reshape(NW, NCHUNK, CHUNK)
    def arrange(vals, fill):
        slots = jnp.concatenate(
            [vals, jnp.full((NTOT * CHUNK - E,), fill, jnp.int32)]
        ).reshape(NTOT, CHUNK)
        a = slots[:16 * M0].reshape(16, M0, CHUNK)
        b = slots[16 * M0:].reshape(16, M1, CHUNK)
        a = jnp.concatenate(
            [a, jnp.full((16, MMAX - M0, CHUNK), fill, jnp.int32)], axis=1)
        b = jnp.concatenate(
            [b, jnp.full((16, MMAX - M1, CHUNK), fill, jnp.int32)], axis=1)
        return jnp.concatenate([a, b], axis=0)

    src_f = arrange(src, 0)
    dst_f = arrange(dst, N)

    pdeg = _build_deg_kernel()(dst3)
    p0 = pdeg[0, :N, 0:1]
    p1 = pdeg[1, :N, 0:1]

    h1 = _matmul(x, W1)
    g1, dinv = _scale0(h1, p0, p1)

    agg = _build_agg_kernel()
    acc1 = agg(g1, src_f, dst_f)
    g2 = _mid(acc1[0, :N], acc1[1, :N], g1, dinv, b1.reshape(1, D), W2)

    acc2 = agg(g2, src_f, dst_f)
    g3 = _mid(acc2[0, :N], acc2[1, :N], g2, dinv, b2.reshape(1, D), W3)

    acc3 = agg(g3, src_f, dst_f)
    return _fin(acc3[0, :N], acc3[1, :N], g3, dinv, b3.reshape(1, D))
